# baseline (speedup 1.0000x reference)
import functools

import jax
import jax.numpy as jnp
from jax import lax
from jax.experimental import pallas as pl
from jax.experimental.pallas import tpu as pltpu
from jax.experimental.pallas import tpu_sc as plsc

N_EMBED = 4096
NUM_EXPERTS = 64
TOP_K = 8
TOKENS = 4 * 4096
BLOCK_T = 512
NUM_CHUNKS = 4
CHUNK_T = TOKENS // NUM_CHUNKS

SC_CORES = 2
SC_SUBCORES = 16
SC_LANES = 16
NUM_WORKERS = SC_CORES * SC_SUBCORES
TOK_PER_W = CHUNK_T // NUM_WORKERS


def _matmul_kernel(x_ref, w_ref, b_ref, out_ref):
    out_ref[...] = jax.lax.dot_general(
        x_ref[...], w_ref[...],
        dimension_numbers=(((1,), (1,)), ((), ())),
        preferred_element_type=jnp.float32,
    ) + b_ref[...]


def _tc_logits(x2d, W, b2d):
    grid = (CHUNK_T // BLOCK_T,)
    return pl.pallas_call(
        _matmul_kernel,
        grid=grid,
        in_specs=[
            pl.BlockSpec((BLOCK_T, N_EMBED), lambda i: (i, 0)),
            pl.BlockSpec((NUM_EXPERTS, N_EMBED), lambda i: (0, 0)),
            pl.BlockSpec((1, NUM_EXPERTS), lambda i: (0, 0)),
        ],
        out_specs=pl.BlockSpec((BLOCK_T, NUM_EXPERTS), lambda i: (i, 0)),
        out_shape=jax.ShapeDtypeStruct((CHUNK_T, NUM_EXPERTS), jnp.float32),
    )(x2d, W, b2d)


def _merge_top16(ak, av, bk, bv):
    rbk = lax.rev(bk, (0,))
    rbv = lax.rev(bv, (0,))
    take_a = ak >= rbk
    mk = jnp.where(take_a, ak, rbk)
    mv = jnp.where(take_a, av, rbv)
    return plsc.sort_key_val(mk, mv, descending=True)


def _sc_body(logits_hbm, out_hbm, idx_hbm, in_v, out_v, idx_v):
    wid = lax.axis_index("s") * SC_CORES + lax.axis_index("c")
    base = wid * TOK_PER_W
    pltpu.sync_copy(logits_hbm.at[pl.ds(base, TOK_PER_W)], in_v)

    lanes = lax.iota(jnp.int32, SC_LANES)
    mask8 = lanes < TOP_K
    zero16 = jnp.zeros((SC_LANES,), jnp.float32)

    @plsc.parallel_loop(0, TOK_PER_W, unroll=8)
    def body(t):
        k0 = in_v[t, pl.ds(0, 16)]
        k1 = in_v[t, pl.ds(16, 16)]
        k2 = in_v[t, pl.ds(32, 16)]
        k3 = in_v[t, pl.ds(48, 16)]
        s0k, s0v = plsc.sort_key_val(k0, lanes, descending=True)
        s1k, s1v = plsc.sort_key_val(k1, lanes + 16, descending=True)
        s2k, s2v = plsc.sort_key_val(k2, lanes + 32, descending=True)
        s3k, s3v = plsc.sort_key_val(k3, lanes + 48, descending=True)
        m01k, m01v = _merge_top16(s0k, s0v, s1k, s1v)
        m23k, m23v = _merge_top16(s2k, s2v, s3k, s3v)
        fk, fv = _merge_top16(m01k, m01v, m23k, m23v)

        e = jnp.exp(fk - jnp.max(fk))
        esel = jnp.where(mask8, e, 0.0)
        probs = esel / jnp.sum(esel)

        out_v[t, pl.ds(0, 16)] = zero16
        out_v[t, pl.ds(16, 16)] = zero16
        out_v[t, pl.ds(32, 16)] = zero16
        out_v[t, pl.ds(48, 16)] = zero16
        tvec = jnp.full((SC_LANES,), t, jnp.int32)
        plsc.store_scatter(out_v, [tvec, fv], probs, mask=mask8)
        plsc.store_scatter(idx_v, [tvec, lanes], fv, mask=mask8)

    pltpu.sync_copy(out_v, out_hbm.at[pl.ds(base, TOK_PER_W)])
    pltpu.sync_copy(idx_v, idx_hbm.at[pl.ds(base, TOK_PER_W)])


_sc_topk = functools.partial(
    pl.kernel,
    mesh=plsc.VectorSubcoreMesh(core_axis_name="c", subcore_axis_name="s"),
    compiler_params=pltpu.CompilerParams(
        needs_layout_passes=False, use_tc_tiling_on_sc=False
    ),
    out_type=[
        jax.ShapeDtypeStruct((CHUNK_T, NUM_EXPERTS), jnp.float32),
        jax.ShapeDtypeStruct((CHUNK_T, TOP_K), jnp.int32),
    ],
    scratch_types=[
        pltpu.VMEM((TOK_PER_W, NUM_EXPERTS), jnp.float32),
        pltpu.VMEM((TOK_PER_W, NUM_EXPERTS), jnp.float32),
        pltpu.VMEM((TOK_PER_W, TOP_K), jnp.int32),
    ],
)(_sc_body)


def kernel(mh_output, W, b):
    B, S, E = mh_output.shape
    x2d = mh_output.reshape(B * S, E)
    b2d = b.reshape(1, NUM_EXPERTS)
    outs = []
    idxs = []
    for c in range(NUM_CHUNKS):
        xc = lax.slice_in_dim(x2d, c * CHUNK_T, (c + 1) * CHUNK_T, axis=0)
        logits_c = _tc_logits(xc, W, b2d)
        out_c, idx_c = _sc_topk(logits_c)
        outs.append(out_c)
        idxs.append(idx_c)
    out = jnp.concatenate(outs, axis=0)
    idx = jnp.concatenate(idxs, axis=0)
    return out.reshape(B, S, NUM_EXPERTS), idx.reshape(B, S, TOP_K)


# speedup vs baseline: 2.0960x; 2.0960x over previous
import functools

import jax
import jax.numpy as jnp
from jax import lax
from jax.experimental import pallas as pl
from jax.experimental.pallas import tpu as pltpu
from jax.experimental.pallas import tpu_sc as plsc

N_EMBED = 4096
NUM_EXPERTS = 64
TOP_K = 8
TOKENS = 4 * 4096
BLOCK_T = 1024

SC_CORES = 2
SC_SUBCORES = 16
SC_LANES = 16
NUM_WORKERS = SC_CORES * SC_SUBCORES
TOK_PER_W = TOKENS // NUM_WORKERS


def _matmul_kernel(x_ref, w_ref, b_ref, out_ref):
    out_ref[...] = jax.lax.dot_general(
        x_ref[...], w_ref[...],
        dimension_numbers=(((1,), (1,)), ((), ())),
        preferred_element_type=jnp.float32,
    ) + b_ref[...]


def _tc_logits(x2d, W, b2d):
    grid = (TOKENS // BLOCK_T,)
    return pl.pallas_call(
        _matmul_kernel,
        grid=grid,
        in_specs=[
            pl.BlockSpec((BLOCK_T, N_EMBED), lambda i: (i, 0)),
            pl.BlockSpec((NUM_EXPERTS, N_EMBED), lambda i: (0, 0)),
            pl.BlockSpec((1, NUM_EXPERTS), lambda i: (0, 0)),
        ],
        out_specs=pl.BlockSpec((BLOCK_T, NUM_EXPERTS), lambda i: (i, 0)),
        out_shape=jax.ShapeDtypeStruct((TOKENS, NUM_EXPERTS), jnp.float32),
    )(x2d, W, b2d)


def _merge_top16(ak, av, bk, bv):
    rbk = lax.rev(bk, (0,))
    rbv = lax.rev(bv, (0,))
    take_a = ak >= rbk
    mk = jnp.where(take_a, ak, rbk)
    mv = jnp.where(take_a, av, rbv)
    return plsc.sort_key_val(mk, mv, descending=True)


def _sc_body(logits_hbm, out_hbm, idx_hbm, in_v, out_v, idx_v):
    wid = lax.axis_index("s") * SC_CORES + lax.axis_index("c")
    base = wid * TOK_PER_W
    pltpu.sync_copy(logits_hbm.at[pl.ds(base, TOK_PER_W)], in_v)

    lanes = lax.iota(jnp.int32, SC_LANES)
    mask8 = lanes < TOP_K
    zero16 = jnp.zeros((SC_LANES,), jnp.float32)

    @plsc.parallel_loop(0, TOK_PER_W, unroll=16)
    def body(t):
        k0 = in_v[t, pl.ds(0, 16)]
        k1 = in_v[t, pl.ds(16, 16)]
        k2 = in_v[t, pl.ds(32, 16)]
        k3 = in_v[t, pl.ds(48, 16)]
        s0k, s0v = plsc.sort_key_val(k0, lanes, descending=True)
        s1k, s1v = plsc.sort_key_val(k1, lanes + 16, descending=True)
        s2k, s2v = plsc.sort_key_val(k2, lanes + 32, descending=True)
        s3k, s3v = plsc.sort_key_val(k3, lanes + 48, descending=True)
        m01k, m01v = _merge_top16(s0k, s0v, s1k, s1v)
        m23k, m23v = _merge_top16(s2k, s2v, s3k, s3v)
        fk, fv = _merge_top16(m01k, m01v, m23k, m23v)

        e = jnp.exp(fk - jnp.max(fk))
        esel = jnp.where(mask8, e, 0.0)
        probs = esel / jnp.sum(esel)

        out_v[t, pl.ds(0, 16)] = zero16
        out_v[t, pl.ds(16, 16)] = zero16
        out_v[t, pl.ds(32, 16)] = zero16
        out_v[t, pl.ds(48, 16)] = zero16
        tvec = jnp.full((SC_LANES,), t, jnp.int32)
        plsc.store_scatter(out_v, [tvec, fv], probs, mask=mask8)
        plsc.store_scatter(idx_v, [tvec, lanes], fv, mask=mask8)

    pltpu.sync_copy(out_v, out_hbm.at[pl.ds(base, TOK_PER_W)])
    pltpu.sync_copy(idx_v, idx_hbm.at[pl.ds(base, TOK_PER_W)])


_sc_topk = functools.partial(
    pl.kernel,
    mesh=plsc.VectorSubcoreMesh(core_axis_name="c", subcore_axis_name="s"),
    compiler_params=pltpu.CompilerParams(
        needs_layout_passes=False, use_tc_tiling_on_sc=False
    ),
    out_type=[
        jax.ShapeDtypeStruct((TOKENS, NUM_EXPERTS), jnp.float32),
        jax.ShapeDtypeStruct((TOKENS, TOP_K), jnp.int32),
    ],
    scratch_types=[
        pltpu.VMEM((TOK_PER_W, NUM_EXPERTS), jnp.float32),
        pltpu.VMEM((TOK_PER_W, NUM_EXPERTS), jnp.float32),
        pltpu.VMEM((TOK_PER_W, TOP_K), jnp.int32),
    ],
)(_sc_body)


def kernel(mh_output, W, b):
    B, S, E = mh_output.shape
    x2d = mh_output.reshape(B * S, E)
    logits = _tc_logits(x2d, W, b.reshape(1, NUM_EXPERTS))
    out, idx = _sc_topk(logits)
    return out.reshape(B, S, NUM_EXPERTS), idx.reshape(B, S, TOP_K)


# speedup vs baseline: 2.1012x; 1.0025x over previous
import functools

import jax
import jax.numpy as jnp
from jax import lax
from jax.experimental import pallas as pl
from jax.experimental.pallas import tpu as pltpu
from jax.experimental.pallas import tpu_sc as plsc

N_EMBED = 4096
NUM_EXPERTS = 64
TOP_K = 8
TOKENS = 4 * 4096
BLOCK_T = 1024

SC_CORES = 2
SC_SUBCORES = 16
SC_LANES = 16
NUM_WORKERS = SC_CORES * SC_SUBCORES
TOK_PER_W = TOKENS // NUM_WORKERS


def _matmul_kernel(x_ref, w_ref, b_ref, out_ref):
    out_ref[...] = jax.lax.dot_general(
        x_ref[...], w_ref[...],
        dimension_numbers=(((1,), (1,)), ((), ())),
        preferred_element_type=jnp.float32,
    ) + b_ref[...]


def _tc_logits(x2d, W, b2d):
    grid = (TOKENS // BLOCK_T,)
    return pl.pallas_call(
        _matmul_kernel,
        grid=grid,
        in_specs=[
            pl.BlockSpec((BLOCK_T, N_EMBED), lambda i: (i, 0)),
            pl.BlockSpec((NUM_EXPERTS, N_EMBED), lambda i: (0, 0)),
            pl.BlockSpec((1, NUM_EXPERTS), lambda i: (0, 0)),
        ],
        out_specs=pl.BlockSpec((BLOCK_T, NUM_EXPERTS), lambda i: (i, 0)),
        out_shape=jax.ShapeDtypeStruct((TOKENS, NUM_EXPERTS), jnp.float32),
    )(x2d, W, b2d)


def _merge_top16(ak, av, bk, bv):
    rbk = lax.rev(bk, (0,))
    rbv = lax.rev(bv, (0,))
    take_a = ak >= rbk
    mk = jnp.where(take_a, ak, rbk)
    mv = jnp.where(take_a, av, rbv)
    return plsc.sort_key_val(mk, mv, descending=True)


def _sc_body(logits_hbm, out_hbm, idx_hbm, in_v, out_v, idx_v):
    wid = lax.axis_index("s") * SC_CORES + lax.axis_index("c")
    base = wid * TOK_PER_W
    pltpu.sync_copy(logits_hbm.at[pl.ds(base, TOK_PER_W)], in_v)

    lanes = lax.iota(jnp.int32, SC_LANES)
    mask8 = lanes < TOP_K
    zero16 = jnp.zeros((SC_LANES,), jnp.float32)

    @plsc.parallel_loop(0, TOK_PER_W, unroll=16)
    def zero_body(t):
        out_v[t, pl.ds(0, 16)] = zero16
        out_v[t, pl.ds(16, 16)] = zero16
        out_v[t, pl.ds(32, 16)] = zero16
        out_v[t, pl.ds(48, 16)] = zero16

    @plsc.parallel_loop(0, TOK_PER_W, unroll=16)
    def body(t):
        k0 = in_v[t, pl.ds(0, 16)]
        k1 = in_v[t, pl.ds(16, 16)]
        k2 = in_v[t, pl.ds(32, 16)]
        k3 = in_v[t, pl.ds(48, 16)]
        s0k, s0v = plsc.sort_key_val(k0, lanes, descending=True)
        s1k, s1v = plsc.sort_key_val(k1, lanes + 16, descending=True)
        s2k, s2v = plsc.sort_key_val(k2, lanes + 32, descending=True)
        s3k, s3v = plsc.sort_key_val(k3, lanes + 48, descending=True)
        m01k, m01v = _merge_top16(s0k, s0v, s1k, s1v)
        m23k, m23v = _merge_top16(s2k, s2v, s3k, s3v)
        fk, fv = _merge_top16(m01k, m01v, m23k, m23v)

        e = jnp.exp(fk - jnp.max(fk))
        esel = jnp.where(mask8, e, 0.0)
        probs = esel / jnp.sum(esel)

        tvec = jnp.full((SC_LANES,), t, jnp.int32)
        plsc.store_scatter(out_v, [tvec, fv], probs, mask=mask8)
        plsc.store_scatter(idx_v, [tvec, lanes], fv, mask=mask8)

    pltpu.sync_copy(out_v, out_hbm.at[pl.ds(base, TOK_PER_W)])
    pltpu.sync_copy(idx_v, idx_hbm.at[pl.ds(base, TOK_PER_W)])


_sc_topk = functools.partial(
    pl.kernel,
    mesh=plsc.VectorSubcoreMesh(core_axis_name="c", subcore_axis_name="s"),
    compiler_params=pltpu.CompilerParams(
        needs_layout_passes=False, use_tc_tiling_on_sc=False
    ),
    out_type=[
        jax.ShapeDtypeStruct((TOKENS, NUM_EXPERTS), jnp.float32),
        jax.ShapeDtypeStruct((TOKENS, TOP_K), jnp.int32),
    ],
    scratch_types=[
        pltpu.VMEM((TOK_PER_W, NUM_EXPERTS), jnp.float32),
        pltpu.VMEM((TOK_PER_W, NUM_EXPERTS), jnp.float32),
        pltpu.VMEM((TOK_PER_W, TOP_K), jnp.int32),
    ],
)(_sc_body)


def kernel(mh_output, W, b):
    B, S, E = mh_output.shape
    x2d = mh_output.reshape(B * S, E)
    logits = _tc_logits(x2d, W, b.reshape(1, NUM_EXPERTS))
    out, idx = _sc_topk(logits)
    return out.reshape(B, S, NUM_EXPERTS), idx.reshape(B, S, TOP_K)


# speedup vs baseline: 2.1087x; 1.0035x over previous
import functools

import jax
import jax.numpy as jnp
from jax import lax
from jax.experimental import pallas as pl
from jax.experimental.pallas import tpu as pltpu
from jax.experimental.pallas import tpu_sc as plsc

N_EMBED = 4096
NUM_EXPERTS = 64
TOP_K = 8
TOKENS = 4 * 4096
BLOCK_T = 512

SC_CORES = 2
SC_SUBCORES = 16
SC_LANES = 16
NUM_WORKERS = SC_CORES * SC_SUBCORES
TOK_PER_W = TOKENS // NUM_WORKERS


def _matmul_kernel(x_ref, w_ref, b_ref, out_ref):
    out_ref[...] = jax.lax.dot_general(
        x_ref[...], w_ref[...],
        dimension_numbers=(((1,), (1,)), ((), ())),
        preferred_element_type=jnp.float32,
    ) + b_ref[...]


def _tc_logits(x2d, W, b2d):
    grid = (TOKENS // BLOCK_T,)
    return pl.pallas_call(
        _matmul_kernel,
        grid=grid,
        in_specs=[
            pl.BlockSpec((BLOCK_T, N_EMBED), lambda i: (i, 0)),
            pl.BlockSpec((NUM_EXPERTS, N_EMBED), lambda i: (0, 0)),
            pl.BlockSpec((1, NUM_EXPERTS), lambda i: (0, 0)),
        ],
        out_specs=pl.BlockSpec((BLOCK_T, NUM_EXPERTS), lambda i: (i, 0)),
        out_shape=jax.ShapeDtypeStruct((TOKENS, NUM_EXPERTS), jnp.float32),
    )(x2d, W, b2d)


def _merge_top16(ak, av, bk, bv):
    rbk = lax.rev(bk, (0,))
    rbv = lax.rev(bv, (0,))
    take_a = ak >= rbk
    mk = jnp.where(take_a, ak, rbk)
    mv = jnp.where(take_a, av, rbv)
    return plsc.sort_key_val(mk, mv, descending=True)


def _sc_body(logits_hbm, out_hbm, idx_hbm, in_v, out_v, idx_v):
    wid = lax.axis_index("s") * SC_CORES + lax.axis_index("c")
    base = wid * TOK_PER_W
    pltpu.sync_copy(logits_hbm.at[pl.ds(base, TOK_PER_W)], in_v)

    lanes = lax.iota(jnp.int32, SC_LANES)
    mask8 = lanes < TOP_K
    zero16 = jnp.zeros((SC_LANES,), jnp.float32)

    @plsc.parallel_loop(0, TOK_PER_W, unroll=8)
    def body(t):
        k0 = in_v[t, pl.ds(0, 16)]
        k1 = in_v[t, pl.ds(16, 16)]
        k2 = in_v[t, pl.ds(32, 16)]
        k3 = in_v[t, pl.ds(48, 16)]
        s0k, s0v = plsc.sort_key_val(k0, lanes, descending=True)
        s1k, s1v = plsc.sort_key_val(k1, lanes + 16, descending=True)
        s2k, s2v = plsc.sort_key_val(k2, lanes + 32, descending=True)
        s3k, s3v = plsc.sort_key_val(k3, lanes + 48, descending=True)
        m01k, m01v = _merge_top16(s0k, s0v, s1k, s1v)
        m23k, m23v = _merge_top16(s2k, s2v, s3k, s3v)
        fk, fv = _merge_top16(m01k, m01v, m23k, m23v)

        e = jnp.exp(fk - jnp.max(fk))
        esel = jnp.where(mask8, e, 0.0)
        probs = esel / jnp.sum(esel)

        out_v[t, pl.ds(0, 16)] = zero16
        out_v[t, pl.ds(16, 16)] = zero16
        out_v[t, pl.ds(32, 16)] = zero16
        out_v[t, pl.ds(48, 16)] = zero16
        tvec = jnp.full((SC_LANES,), t, jnp.int32)
        plsc.store_scatter(out_v, [tvec, fv], probs, mask=mask8)
        plsc.store_scatter(idx_v, [tvec, lanes], fv, mask=mask8)

    pltpu.sync_copy(out_v, out_hbm.at[pl.ds(base, TOK_PER_W)])
    pltpu.sync_copy(idx_v, idx_hbm.at[pl.ds(base, TOK_PER_W)])


_sc_topk = functools.partial(
    pl.kernel,
    mesh=plsc.VectorSubcoreMesh(core_axis_name="c", subcore_axis_name="s"),
    compiler_params=pltpu.CompilerParams(
        needs_layout_passes=False, use_tc_tiling_on_sc=False
    ),
    out_type=[
        jax.ShapeDtypeStruct((TOKENS, NUM_EXPERTS), jnp.float32),
        jax.ShapeDtypeStruct((TOKENS, TOP_K), jnp.int32),
    ],
    scratch_types=[
        pltpu.VMEM((TOK_PER_W, NUM_EXPERTS), jnp.float32),
        pltpu.VMEM((TOK_PER_W, NUM_EXPERTS), jnp.float32),
        pltpu.VMEM((TOK_PER_W, TOP_K), jnp.int32),
    ],
)(_sc_body)


def kernel(mh_output, W, b):
    B, S, E = mh_output.shape
    x2d = mh_output.reshape(B * S, E)
    logits = _tc_logits(x2d, W, b.reshape(1, NUM_EXPERTS))
    out, idx = _sc_topk(logits)
    return out.reshape(B, S, NUM_EXPERTS), idx.reshape(B, S, TOP_K)
